# baseline (speedup 1.0000x reference)
import functools

import jax
import jax.numpy as jnp
import numpy as np
from jax import lax
from jax.experimental import pallas as pl
from jax.experimental.pallas import tpu as pltpu
from jax.experimental.pallas import tpu_sc as plsc

_NUM_FIELDS = 26
_FIELD_DIM = 100000
_EMBED_DIM = 16
_BATCH = 16384
_TOTAL = _NUM_FIELDS * _FIELD_DIM
_PAIR_I, _PAIR_J = np.triu_indices(_NUM_FIELDS, k=1)
_NUM_PAIRS = _PAIR_I.shape[0]

_NW = 32
_CHUNK = 64
_ROWS = _CHUNK * _NUM_FIELDS
_IDX_ROWS = _ROWS // 128
_ELEMS_PER_W = _BATCH // _NW
_CHUNKS_PER_W = _ELEMS_PER_W // _CHUNK
_XO_ROWS = _BATCH * _NUM_FIELDS // 128


_GATHER_DNUMS = lax.GatherDimensionNumbers(
    offset_dims=(), collapsed_slice_dims=(0,), start_index_map=(0,))


def _lane_shuffle(v, idx):
    return lax.gather(v, idx[:, None], _GATHER_DNUMS, slice_sizes=(1,),
                      mode=lax.GatherScatterMode.PROMISE_IN_BOUNDS)


def _tr_body(in_ref, out_ref):
    x = in_ref[...]
    out_ref[...] = jnp.swapaxes(x, 0, 1)


@jax.jit
def _emb_to_rows(emb_t):
    return pl.pallas_call(
        _tr_body,
        grid=(318,),
        in_specs=[pl.BlockSpec((16, 8192), lambda i: (0, i))],
        out_specs=pl.BlockSpec((8192, 16), lambda i: (i, 0)),
        out_shape=jax.ShapeDtypeStruct((_TOTAL, _EMBED_DIM), jnp.float32),
    )(emb_t)


def _fwfm_body(xo_hbm, emb_hbm, lin_hbm, rb_hbm, out_hbm,
               idx_v, rows_v, w_v, rb_v, out_v, sem_e, sem_w):
    wid = lax.axis_index("s") * 2 + lax.axis_index("c")
    pltpu.sync_copy(rb_hbm, rb_v)
    pltpu.sync_copy(
        xo_hbm.at[pl.ds(wid * (_IDX_ROWS * _CHUNKS_PER_W),
                        _IDX_ROWS * _CHUNKS_PER_W)], idx_v)

    def chunk_body(c, _):
        cps = []
        for j in range(_IDX_ROWS):
            isl = idx_v.at[c * _IDX_ROWS + j]
            cps.append(pltpu.async_copy(
                emb_hbm.at[isl],
                rows_v.at[pl.ds(j * 128, 128)], sem_e))
            cps.append(pltpu.async_copy(
                lin_hbm.at[isl],
                w_v.at[pl.ds(j * 128, 128)], sem_w))
        for cp in cps:
            cp.wait()

        lanes = lax.iota(jnp.int32, 16)

        def group_body(g, _):
            def elem_body(k, res):
                e = g * 16 + k
                row0 = e * _NUM_FIELDS
                vs = [rows_v[row0 + f] for f in range(_NUM_FIELDS)]
                accs = [jnp.zeros((16,), jnp.float32) for _ in range(8)]
                for p in range(_NUM_PAIRS):
                    i, j = int(_PAIR_I[p]), int(_PAIR_J[p])
                    accs[p % 8] = accs[p % 8] + rb_v[p] * (vs[i] * vs[j])
                a = (accs[0] + accs[1]) + (accs[2] + accs[3])
                b = (accs[4] + accs[5]) + (accs[6] + accs[7])
                wv1 = w_v[pl.ds(row0, 16)]
                wv2 = w_v[pl.ds(row0 + 16, 16)]
                wv2 = jnp.where(lanes < _NUM_FIELDS - 16, wv2, 0.0)
                v = (a + b) + (wv1 + wv2)
                for m in (8, 4, 2, 1):
                    v = v + _lane_shuffle(v, lanes ^ m)
                return jnp.where(lanes == k, v, res)

            res = lax.fori_loop(0, 16, elem_body,
                                jnp.zeros((16,), jnp.float32))
            out_v[pl.ds(g * 16, 16)] = res
            return 0

        lax.fori_loop(0, _CHUNK // 16, group_body, 0)

        pltpu.sync_copy(
            out_v, out_hbm.at[pl.ds(wid * _ELEMS_PER_W + c * _CHUNK, _CHUNK)])
        return 0

    lax.fori_loop(0, _CHUNKS_PER_W, chunk_body, 0)


@jax.jit
def _fwfm_sc(xo2d, embed_table, linear_w, rb):
    mesh = plsc.VectorSubcoreMesh(core_axis_name="c", subcore_axis_name="s")
    return pl.kernel(
        _fwfm_body,
        out_type=jax.ShapeDtypeStruct((_BATCH,), jnp.float32),
        mesh=mesh,
        compiler_params=pltpu.CompilerParams(use_tc_tiling_on_sc=False),
        scratch_types=[
            pltpu.VMEM((_IDX_ROWS * _CHUNKS_PER_W, 128), jnp.int32),
            pltpu.VMEM((_ROWS, _EMBED_DIM), jnp.float32),
            pltpu.VMEM((_ROWS + 16,), jnp.float32),
            pltpu.VMEM((_NUM_PAIRS, 16), jnp.float32),
            pltpu.VMEM((_CHUNK,), jnp.float32),
            pltpu.SemaphoreType.DMA,
            pltpu.SemaphoreType.DMA,
        ],
    )(xo2d, embed_table, linear_w, rb)


def kernel(x, embed_table, linear_w, linear_bias, pair_weights):
    offsets = jnp.asarray(np.arange(_NUM_FIELDS) * _FIELD_DIM, dtype=x.dtype)
    xo2d = (x + offsets[None, :]).reshape(_XO_ROWS, 128)
    rb = jnp.broadcast_to(pair_weights[:, None], (_NUM_PAIRS, 16))
    emb_rows = _emb_to_rows(embed_table.T)
    out = _fwfm_sc(xo2d, emb_rows, linear_w[:, 0], rb)
    return out[:, None] + linear_bias[None, :]


# speedup vs baseline: 1.7472x; 1.7472x over previous
import jax
import jax.numpy as jnp
import numpy as np
from jax import lax
from jax.experimental import pallas as pl
from jax.experimental.pallas import tpu as pltpu
from jax.experimental.pallas import tpu_sc as plsc

_NUM_FIELDS = 26
_FIELD_DIM = 100000
_EMBED_DIM = 16
_BATCH = 16384
_TOTAL = _NUM_FIELDS * _FIELD_DIM
_PAIR_I, _PAIR_J = np.triu_indices(_NUM_FIELDS, k=1)
_NUM_PAIRS = _PAIR_I.shape[0]

_NW = 32
_FPAD = 32
_CHUNK = 8
_LPC = _CHUNK * _FPAD
_ELEMS_PER_W = _BATCH // _NW
_CHUNKS_PER_W = _ELEMS_PER_W // _CHUNK
_IDX_ROWS_PER_W = _ELEMS_PER_W * _FPAD // 128

_GATHER_DNUMS = lax.GatherDimensionNumbers(
    offset_dims=(), collapsed_slice_dims=(0,), start_index_map=(0,))


def _lane_shuffle(v, idx):
    return lax.gather(v, idx[:, None], _GATHER_DNUMS, slice_sizes=(1,),
                      mode=lax.GatherScatterMode.PROMISE_IN_BOUNDS)


_EYE16 = np.eye(16, dtype=np.float32)
_EMERGE = np.zeros((8, 16, 128), dtype=np.float32)
for _rl in range(8):
    for _d in range(16):
        _EMERGE[_rl, _d, _rl * 16 + _d] = 1.0


def _tr_body(in_ref, lin_ref, out_ref, lout_ref):
    x = in_ref[...]
    y = jnp.swapaxes(x, 0, 1)
    y3 = y.reshape(1024, 8, 16)
    for rl in range(8):
        out_ref[:, pl.ds(rl * 16, 16)] = y3[:, rl, :]
    lout_ref[...] = lin_ref[0, :]


@jax.jit
def _emb_to_lines(emb_t, lin_t):
    return pl.pallas_call(
        _tr_body,
        grid=(318,),
        in_specs=[pl.BlockSpec((16, 8192), lambda i: (0, i)),
                  pl.BlockSpec((1, 8192), lambda i: (0, i))],
        out_specs=[pl.BlockSpec((1024, 128), lambda i: (i, 0)),
                   pl.BlockSpec((8192,), lambda i: (i,))],
        out_shape=(jax.ShapeDtypeStruct((_TOTAL // 8, 128), jnp.float32),
                   jax.ShapeDtypeStruct((_TOTAL,), jnp.float32)),
    )(emb_t, lin_t)


def _fwfm_body(xo_hbm, emb_hbm, lin_hbm, rb_hbm, out_hbm,
               idx_v, li_v, lines0, lines1, w0, w1, rb_v, out_v,
               sem_e, sem_w):
    wid = lax.axis_index("s") * 2 + lax.axis_index("c")
    pltpu.sync_copy(rb_hbm, rb_v)
    pltpu.sync_copy(
        xo_hbm.at[pl.ds(wid * _IDX_ROWS_PER_W, _IDX_ROWS_PER_W)], idx_v)

    def li_row(r, _):
        for v8 in range(8):
            li_v[r, pl.ds(v8 * 16, 16)] = jnp.right_shift(
                idx_v[r, pl.ds(v8 * 16, 16)], 3)
        return 0

    lax.fori_loop(0, _IDX_ROWS_PER_W, li_row, 0)

    lanes = lax.iota(jnp.int32, 16)
    lines_bufs = (lines0, lines1)
    w_bufs = (w0, w1)

    def fire(c, buf):
        lines_v, w_v = lines_bufs[buf], w_bufs[buf]
        cps = []
        for q in range(2):
            r = c * 2 + q
            cps.append(pltpu.async_copy(
                emb_hbm.at[li_v.at[r]],
                lines_v.at[pl.ds(q * 128, 128)], sem_e))
            cps.append(pltpu.async_copy(
                lin_hbm.at[idx_v.at[r]],
                w_v.at[pl.ds(q * 128, 128)], sem_w))
        return cps

    def compute(c, buf, half, res):
        lines_v, w_v = lines_bufs[buf], w_bufs[buf]

        def elem_body(e, res):
            row0 = e * _FPAD
            soa = (idx_v[c * 2 + e // 4,
                         pl.ds((e % 4) * _FPAD, 16)] & 7) * 16
            sob = (idx_v[c * 2 + e // 4,
                         pl.ds((e % 4) * _FPAD + 16, 16)] & 7) * 16
            vs = []
            for f in range(_NUM_FIELDS):
                s = soa[f] if f < 16 else sob[f - 16]
                vs.append(lines_v[row0 + f, pl.ds(s, 16)])
            accs = [jnp.zeros((16,), jnp.float32) for _ in range(8)]
            for p in range(_NUM_PAIRS):
                i, j = int(_PAIR_I[p]), int(_PAIR_J[p])
                rbp = rb_v[p // 8, pl.ds((p % 8) * 16, 16)]
                accs[p % 8] = accs[p % 8] + rbp * (vs[i] * vs[j])
            a = (accs[0] + accs[1]) + (accs[2] + accs[3])
            b = (accs[4] + accs[5]) + (accs[6] + accs[7])
            wv1 = w_v[pl.ds(row0, 16)]
            wv2 = w_v[pl.ds(row0 + 16, 16)]
            wv2 = jnp.where(lanes < _NUM_FIELDS - 16, wv2, 0.0)
            v = (a + b) + (wv1 + wv2)
            for m in (8, 4, 2, 1):
                v = v + _lane_shuffle(v, lanes ^ m)
            lane = half * _CHUNK + e
            return jnp.where(lanes == lane, v, res)

        return lax.fori_loop(0, _CHUNK, elem_body, res)

    def drain(buf):
        pltpu.make_async_copy(
            emb_hbm.at[li_v.at[0]], lines_bufs[buf].at[pl.ds(0, 128)],
            sem_e).wait()
        pltpu.make_async_copy(
            emb_hbm.at[li_v.at[0]], lines_bufs[buf].at[pl.ds(128, 128)],
            sem_e).wait()
        pltpu.make_async_copy(
            lin_hbm.at[idx_v.at[0]], w_bufs[buf].at[pl.ds(0, 128)],
            sem_w).wait()
        pltpu.make_async_copy(
            lin_hbm.at[idx_v.at[0]], w_bufs[buf].at[pl.ds(128, 128)],
            sem_w).wait()

    fire(0, 0)

    def pair_body(t, _):
        c0 = t * 2
        drain(0)
        fire(c0 + 1, 1)
        res = compute(c0, 0, 0, jnp.zeros((16,), jnp.float32))
        drain(1)

        @pl.when(c0 + 2 < _CHUNKS_PER_W)
        def _():
            fire(c0 + 2, 0)

        res = compute(c0 + 1, 1, 1, res)
        out_v[pl.ds(t * 16, 16)] = res
        return 0

    lax.fori_loop(0, _CHUNKS_PER_W // 2, pair_body, 0)
    pltpu.sync_copy(out_v, out_hbm.at[pl.ds(wid * _ELEMS_PER_W,
                                            _ELEMS_PER_W)])


@jax.jit
def _fwfm_sc(xo2d, emb_lines, lin1d, rb):
    mesh = plsc.VectorSubcoreMesh(core_axis_name="c", subcore_axis_name="s")
    return pl.kernel(
        _fwfm_body,
        out_type=jax.ShapeDtypeStruct((_BATCH,), jnp.float32),
        mesh=mesh,
        compiler_params=pltpu.CompilerParams(use_tc_tiling_on_sc=False),
        scratch_types=[
            pltpu.VMEM((_IDX_ROWS_PER_W, 128), jnp.int32),
            pltpu.VMEM((_IDX_ROWS_PER_W, 128), jnp.int32),
            pltpu.VMEM((_LPC, 128), jnp.float32),
            pltpu.VMEM((_LPC, 128), jnp.float32),
            pltpu.VMEM((_LPC,), jnp.float32),
            pltpu.VMEM((_LPC,), jnp.float32),
            pltpu.VMEM((41, 128), jnp.float32),
            pltpu.VMEM((_ELEMS_PER_W,), jnp.float32),
            pltpu.SemaphoreType.DMA,
            pltpu.SemaphoreType.DMA,
        ],
    )(xo2d, emb_lines, lin1d, rb)


def kernel(x, embed_table, linear_w, linear_bias, pair_weights):
    offsets = jnp.asarray(np.arange(_NUM_FIELDS) * _FIELD_DIM, dtype=x.dtype)
    xo = x + offsets[None, :]
    xo_pad = jnp.concatenate(
        [xo] + [xo[:, _NUM_FIELDS - 1:]] * (_FPAD - _NUM_FIELDS), axis=1)
    xo2d = xo_pad.reshape(_BATCH * _FPAD // 128, 128)
    rbp = jnp.zeros((328,), jnp.float32).at[:_NUM_PAIRS].set(pair_weights)
    rb = jnp.repeat(rbp[:, None], 16, axis=1).reshape(41, 128)
    emb_lines, lin1d = _emb_to_lines(embed_table.T, linear_w.T)
    out = _fwfm_sc(xo2d, emb_lines, lin1d, rb)
    return out[:, None] + linear_bias[None, :]


# speedup vs baseline: 1.8006x; 1.0306x over previous
import jax
import jax.numpy as jnp
import numpy as np
from jax import lax
from jax.experimental import pallas as pl
from jax.experimental.pallas import tpu as pltpu
from jax.experimental.pallas import tpu_sc as plsc

_NUM_FIELDS = 26
_FIELD_DIM = 100000
_EMBED_DIM = 16
_BATCH = 16384
_TOTAL = _NUM_FIELDS * _FIELD_DIM
_PAIR_I, _PAIR_J = np.triu_indices(_NUM_FIELDS, k=1)
_NUM_PAIRS = _PAIR_I.shape[0]

_NW = 32
_FPAD = 32
_CHUNK = 8
_LPC = _CHUNK * _FPAD
_ELEMS_PER_W = _BATCH // _NW
_CHUNKS_PER_W = _ELEMS_PER_W // _CHUNK
_IDX_ROWS_PER_W = _ELEMS_PER_W * _FPAD // 128

_GATHER_DNUMS = lax.GatherDimensionNumbers(
    offset_dims=(), collapsed_slice_dims=(0,), start_index_map=(0,))


def _lane_shuffle(v, idx):
    return lax.gather(v, idx[:, None], _GATHER_DNUMS, slice_sizes=(1,),
                      mode=lax.GatherScatterMode.PROMISE_IN_BOUNDS)


_EYE16 = np.eye(16, dtype=np.float32)
_EMERGE = np.zeros((8, 16, 128), dtype=np.float32)
for _rl in range(8):
    for _d in range(16):
        _EMERGE[_rl, _d, _rl * 16 + _d] = 1.0


def _tr_body(in_ref, lin_ref, out_ref, lout_ref):
    x = in_ref[...]
    y = jnp.swapaxes(x, 0, 1)
    y3 = y.reshape(1024, 8, 16)
    for rl in range(8):
        out_ref[:, pl.ds(rl * 16, 16)] = y3[:, rl, :]
    lout_ref[...] = lin_ref[0, :]


@jax.jit
def _emb_to_lines(emb_t, lin_t):
    return pl.pallas_call(
        _tr_body,
        grid=(318,),
        in_specs=[pl.BlockSpec((16, 8192), lambda i: (0, i)),
                  pl.BlockSpec((1, 8192), lambda i: (0, i))],
        out_specs=[pl.BlockSpec((1024, 128), lambda i: (i, 0)),
                   pl.BlockSpec((8192,), lambda i: (i,))],
        out_shape=(jax.ShapeDtypeStruct((_TOTAL // 8, 128), jnp.float32),
                   jax.ShapeDtypeStruct((_TOTAL,), jnp.float32)),
    )(emb_t, lin_t)


def _fwfm_body(xo_hbm, emb_hbm, lin_hbm, rb_hbm, out_hbm,
               idx_v, li_v, lines0, lines1, w0, w1, rb_v, out_v,
               sem_e, sem_w):
    wid = lax.axis_index("s") * 2 + lax.axis_index("c")
    pltpu.sync_copy(rb_hbm, rb_v)
    pltpu.sync_copy(
        xo_hbm.at[pl.ds(wid * _IDX_ROWS_PER_W, _IDX_ROWS_PER_W)], idx_v)

    def li_row(r, _):
        for v8 in range(8):
            li_v[r, pl.ds(v8 * 16, 16)] = jnp.right_shift(
                idx_v[r, pl.ds(v8 * 16, 16)], 3)
        return 0

    lax.fori_loop(0, _IDX_ROWS_PER_W, li_row, 0)

    lanes = lax.iota(jnp.int32, 16)
    lines_bufs = (lines0, lines1)
    w_bufs = (w0, w1)

    def fire(c, buf):
        lines_v, w_v = lines_bufs[buf], w_bufs[buf]
        cps = []
        for q in range(2):
            r = c * 2 + q
            cps.append(pltpu.async_copy(
                emb_hbm.at[li_v.at[r]],
                lines_v.at[pl.ds(q * 128, 128)], sem_e))
            cps.append(pltpu.async_copy(
                lin_hbm.at[idx_v.at[r]],
                w_v.at[pl.ds(q * 128, 128)], sem_w))
        return cps

    def compute(c, buf, half, res):
        lines_v, w_v = lines_bufs[buf], w_bufs[buf]

        def load_fields(e):
            row0 = e * _FPAD
            soa = (idx_v[c * 2 + e // 4,
                         pl.ds((e % 4) * _FPAD, 16)] & 7) * 16
            sob = (idx_v[c * 2 + e // 4,
                         pl.ds((e % 4) * _FPAD + 16, 16)] & 7) * 16
            vs = []
            for f in range(_NUM_FIELDS):
                s = soa[f] if f < 16 else sob[f - 16]
                vs.append(lines_v[row0 + f, pl.ds(s, 16)])
            return vs

        def finish(accs, e):
            a = (accs[0] + accs[1]) + (accs[2] + accs[3])
            row0 = e * _FPAD
            wv1 = w_v[pl.ds(row0, 16)]
            wv2 = w_v[pl.ds(row0 + 16, 16)]
            wv2 = jnp.where(lanes < _NUM_FIELDS - 16, wv2, 0.0)
            v = a + (wv1 + wv2)
            for m in (8, 4, 2, 1):
                v = v + _lane_shuffle(v, lanes ^ m)
            return v

        def elem_body(e2, res):
            ea = e2 * 2
            eb = ea + 1
            vsa = load_fields(ea)
            vsb = load_fields(eb)
            za = [jnp.zeros((16,), jnp.float32) for _ in range(4)]
            zb = [jnp.zeros((16,), jnp.float32) for _ in range(4)]
            for p in range(_NUM_PAIRS):
                i, j = int(_PAIR_I[p]), int(_PAIR_J[p])
                rbp = rb_v[p // 8, pl.ds((p % 8) * 16, 16)]
                za[p % 4] = za[p % 4] + rbp * (vsa[i] * vsa[j])
                zb[p % 4] = zb[p % 4] + rbp * (vsb[i] * vsb[j])
            va = finish(za, ea)
            vb = finish(zb, eb)
            res = jnp.where(lanes == half * _CHUNK + ea, va, res)
            return jnp.where(lanes == half * _CHUNK + eb, vb, res)

        return lax.fori_loop(0, _CHUNK // 2, elem_body, res)

    def drain(buf):
        pltpu.make_async_copy(
            emb_hbm.at[li_v.at[0]], lines_bufs[buf].at[pl.ds(0, 128)],
            sem_e).wait()
        pltpu.make_async_copy(
            emb_hbm.at[li_v.at[0]], lines_bufs[buf].at[pl.ds(128, 128)],
            sem_e).wait()
        pltpu.make_async_copy(
            lin_hbm.at[idx_v.at[0]], w_bufs[buf].at[pl.ds(0, 128)],
            sem_w).wait()
        pltpu.make_async_copy(
            lin_hbm.at[idx_v.at[0]], w_bufs[buf].at[pl.ds(128, 128)],
            sem_w).wait()

    fire(0, 0)

    def pair_body(t, _):
        c0 = t * 2
        drain(0)
        fire(c0 + 1, 1)
        res = compute(c0, 0, 0, jnp.zeros((16,), jnp.float32))
        drain(1)

        @pl.when(c0 + 2 < _CHUNKS_PER_W)
        def _():
            fire(c0 + 2, 0)

        res = compute(c0 + 1, 1, 1, res)
        out_v[pl.ds(t * 16, 16)] = res
        return 0

    lax.fori_loop(0, _CHUNKS_PER_W // 2, pair_body, 0)
    pltpu.sync_copy(out_v, out_hbm.at[pl.ds(wid * _ELEMS_PER_W,
                                            _ELEMS_PER_W)])


@jax.jit
def _fwfm_sc(xo2d, emb_lines, lin1d, rb):
    mesh = plsc.VectorSubcoreMesh(core_axis_name="c", subcore_axis_name="s")
    return pl.kernel(
        _fwfm_body,
        out_type=jax.ShapeDtypeStruct((_BATCH,), jnp.float32),
        mesh=mesh,
        compiler_params=pltpu.CompilerParams(use_tc_tiling_on_sc=False),
        scratch_types=[
            pltpu.VMEM((_IDX_ROWS_PER_W, 128), jnp.int32),
            pltpu.VMEM((_IDX_ROWS_PER_W, 128), jnp.int32),
            pltpu.VMEM((_LPC, 128), jnp.float32),
            pltpu.VMEM((_LPC, 128), jnp.float32),
            pltpu.VMEM((_LPC,), jnp.float32),
            pltpu.VMEM((_LPC,), jnp.float32),
            pltpu.VMEM((41, 128), jnp.float32),
            pltpu.VMEM((_ELEMS_PER_W,), jnp.float32),
            pltpu.SemaphoreType.DMA,
            pltpu.SemaphoreType.DMA,
        ],
    )(xo2d, emb_lines, lin1d, rb)


def kernel(x, embed_table, linear_w, linear_bias, pair_weights):
    offsets = jnp.asarray(np.arange(_NUM_FIELDS) * _FIELD_DIM, dtype=x.dtype)
    xo = x + offsets[None, :]
    xo_pad = jnp.concatenate(
        [xo] + [xo[:, _NUM_FIELDS - 1:]] * (_FPAD - _NUM_FIELDS), axis=1)
    xo2d = xo_pad.reshape(_BATCH * _FPAD // 128, 128)
    rbp = jnp.zeros((328,), jnp.float32).at[:_NUM_PAIRS].set(pair_weights)
    rb = jnp.repeat(rbp[:, None], 16, axis=1).reshape(41, 128)
    emb_lines, lin1d = _emb_to_lines(embed_table.T, linear_w.T)
    out = _fwfm_sc(xo2d, emb_lines, lin1d, rb)
    return out[:, None] + linear_bias[None, :]
